# baseline (speedup 1.0000x reference)
import jax
import jax.numpy as jnp
from jax.experimental import pallas as pl

K = 7
B = 4096
C = 1000
BB = 256


def _majority_kernel(x_ref, out_ref):
    x = x_ref[...]
    am = jnp.argmax(x, axis=2).astype(jnp.int32)
    rows = [am[i] for i in range(K)]
    keys = []
    for j in range(K):
        cnt = (rows[0] == rows[j]).astype(jnp.int32)
        for i in range(1, K):
            cnt = cnt + (rows[i] == rows[j]).astype(jnp.int32)
        keys.append(cnt * 1024 - rows[j])
    best = keys[0]
    for j in range(1, K):
        best = jnp.maximum(best, keys[j])
    idx = jnp.where(keys[0] == best, 0, -1)
    for j in range(1, K):
        idx = jnp.maximum(idx, jnp.where(keys[j] == best, j, -1))
    out_ref[...] = idx.astype(jnp.int32)


def kernel(scores):
    grid = (B // BB,)
    out = pl.pallas_call(
        _majority_kernel,
        grid=grid,
        in_specs=[pl.BlockSpec((K, BB, C), lambda i: (0, i, 0))],
        out_specs=pl.BlockSpec((BB,), lambda i: (i,)),
        out_shape=jax.ShapeDtypeStruct((B,), jnp.int32),
    )(scores)
    return out


# speedup vs baseline: 4.0483x; 4.0483x over previous
import jax
import jax.numpy as jnp
from jax.experimental import pallas as pl

K = 7
B = 4096
C = 1000
BB = 512

GRID = (B // BB,)
IN_SPECS = [pl.BlockSpec((K, C, BB), lambda i: (0, 0, i))]
OUT_SPECS = pl.BlockSpec((BB,), lambda i: (i,))
OUT_SHAPE = jax.ShapeDtypeStruct((B,), jnp.int32)


def _majority_kernel(x_ref, out_ref):
    x = x_ref[...]
    am = jnp.argmax(x, axis=1).astype(jnp.int32)
    rows = [am[i] for i in range(K)]
    keys = []
    for j in range(K):
        cnt = (rows[0] == rows[j]).astype(jnp.int32)
        for i in range(1, K):
            cnt = cnt + (rows[i] == rows[j]).astype(jnp.int32)
        keys.append(cnt * 1024 - rows[j])
    best = keys[0]
    for j in range(1, K):
        best = jnp.maximum(best, keys[j])
    idx = jnp.where(keys[0] == best, 0, -1)
    for j in range(1, K):
        idx = jnp.maximum(idx, jnp.where(keys[j] == best, j, -1))
    out_ref[...] = idx.astype(jnp.int32)


def kernel(scores):
    st = jnp.transpose(scores, (0, 2, 1))
    out = pl.pallas_call(
        _majority_kernel,
        grid=GRID,
        in_specs=IN_SPECS,
        out_specs=OUT_SPECS,
        out_shape=OUT_SHAPE,
    )(st)
    return out


# speedup vs baseline: 4.1686x; 1.0297x over previous
import jax
import jax.numpy as jnp
from jax.experimental import pallas as pl

K = 7
B = 4096
C = 1000
BB = 256

GRID = (B // BB,)
IN_SPECS = [pl.BlockSpec((K, C, BB), lambda i: (0, 0, i))]
OUT_SPECS = pl.BlockSpec((BB,), lambda i: (i,))
OUT_SHAPE = jax.ShapeDtypeStruct((B,), jnp.int32)


def _majority_kernel(x_ref, out_ref):
    x = x_ref[...]
    am = jnp.argmax(x, axis=1).astype(jnp.int32)
    rows = [am[i] for i in range(K)]
    keys = []
    for j in range(K):
        cnt = (rows[0] == rows[j]).astype(jnp.int32)
        for i in range(1, K):
            cnt = cnt + (rows[i] == rows[j]).astype(jnp.int32)
        keys.append(cnt * 1024 - rows[j])
    best = keys[0]
    for j in range(1, K):
        best = jnp.maximum(best, keys[j])
    idx = jnp.where(keys[0] == best, 0, -1)
    for j in range(1, K):
        idx = jnp.maximum(idx, jnp.where(keys[j] == best, j, -1))
    out_ref[...] = idx.astype(jnp.int32)


def kernel(scores):
    st = jnp.transpose(scores, (0, 2, 1))
    out = pl.pallas_call(
        _majority_kernel,
        grid=GRID,
        in_specs=IN_SPECS,
        out_specs=OUT_SPECS,
        out_shape=OUT_SHAPE,
    )(st)
    return out
